# baseline (speedup 1.0000x reference)
import jax
import jax.numpy as jnp
from jax import lax
from jax.experimental import pallas as pl
from jax.experimental.pallas import tpu as pltpu
from jax.experimental.pallas import tpu_sc as plsc

VOCAB = 1000000
MAX_TOKENS = 1000
EMBED_DIM = 32
BATCH = 16384
SEQ = 20

NUM_CORES = 2
NUM_SUBCORES = 16
LANES = 16
NUM_WORKERS = NUM_CORES * NUM_SUBCORES
B_PER_W = BATCH // NUM_WORKERS
ROW_CHUNKS = B_PER_W // LANES
TILE_W = 128
SUB_W = 128
PIPE = 12


def _body(tid_hbm, tokT_hbm, tabT_hbm, ttabT_hbm, outT_hbm,
          tid_v, blk_v, titleT_v, ttab_v, tok_v, poolT_v, gsems):
    wid = lax.axis_index("s") * NUM_CORES + lax.axis_index("c")
    base = wid * B_PER_W

    pltpu.sync_copy(tid_hbm.at[pl.ds(base, B_PER_W)], tid_v)
    pltpu.sync_copy(tokT_hbm.at[:, pl.ds(base, B_PER_W)], tok_v)
    pltpu.sync_copy(ttabT_hbm, ttab_v)

    iota = lax.iota(jnp.int32, LANES)
    zeros16 = jnp.zeros((LANES,), jnp.float32)

    plsc.store_scatter(ttab_v, [iota, jnp.zeros((LANES,), jnp.int32)], zeros16)
    plsc.store_scatter(ttab_v, [iota + LANES, jnp.zeros((LANES,), jnp.int32)],
                       zeros16)

    def _col_copy(col8, slot):
        c0 = pl.multiple_of(col8, SUB_W)
        return pltpu.make_async_copy(
            tabT_hbm.at[:, pl.ds(c0, SUB_W)], blk_v.at[slot], gsems.at[slot])

    def title_body(c, carry):
        idvec = tid_v[pl.ds(pl.multiple_of(c * LANES, LANES), LANES)]
        col8 = (idvec // SUB_W) * SUB_W
        off8 = lax.rem(idvec, SUB_W)
        for l in range(PIPE):
            _col_copy(col8[l], l).start()
        for l in range(LANES):
            slot = l % PIPE
            _col_copy(col8[l], slot).wait()
            off = jnp.full((LANES,), off8[l], jnp.int32)
            v0 = plsc.load_gather(blk_v.at[slot], [iota, off])
            v1 = plsc.load_gather(blk_v.at[slot], [iota + LANES, off])
            if l + PIPE < LANES:
                _col_copy(col8[l + PIPE], slot).start()
            icol = jnp.full((LANES,), c * LANES + l, jnp.int32)
            plsc.store_scatter(titleT_v, [iota, icol], v0)
            plsc.store_scatter(titleT_v, [iota + LANES, icol], v1)
        return carry

    lax.fori_loop(0, ROW_CHUNKS, title_body, 0)

    def _tree_sum(vals):
        while len(vals) > 1:
            nxt = [a + b for a, b in zip(vals[::2], vals[1::2])]
            if len(vals) % 2:
                nxt.append(vals[-1])
            vals = nxt
        return vals[0]

    def chunk_body(j, carry):
        row_idx = j * LANES + iota
        tks = [
            plsc.load_gather(tok_v, [jnp.full((LANES,), l, jnp.int32), row_idx])
            for l in range(SEQ)
        ]
        cnt = _tree_sum([(tk != 0).astype(jnp.float32) for tk in tks])
        rcp = 1.0 / jnp.maximum(cnt, 1e-9)
        for d in range(EMBED_DIM):
            drow = jnp.full((LANES,), d, jnp.int32)
            s = _tree_sum([plsc.load_gather(ttab_v, [drow, tk]) for tk in tks])
            plsc.store_scatter(poolT_v, [drow, row_idx], s * rcp)
        return carry

    lax.fori_loop(0, ROW_CHUNKS, chunk_body, 0)

    pltpu.sync_copy(titleT_v, outT_hbm.at[pl.ds(0, EMBED_DIM), pl.ds(base, B_PER_W)])
    pltpu.sync_copy(poolT_v, outT_hbm.at[pl.ds(EMBED_DIM, EMBED_DIM), pl.ds(base, B_PER_W)])


@jax.jit
def kernel(title_ids, token_ids, title_table, token_table):
    tokT = token_ids.T
    tabT = title_table.T
    ttabT = token_table.T

    run = pl.kernel(
        _body,
        out_type=jax.ShapeDtypeStruct((2 * EMBED_DIM, BATCH), jnp.float32),
        mesh=plsc.VectorSubcoreMesh(core_axis_name="c", subcore_axis_name="s"),
        compiler_params=pltpu.CompilerParams(
            needs_layout_passes=False, use_tc_tiling_on_sc=True),
        scratch_types=[
            pltpu.VMEM((B_PER_W,), jnp.int32),
            pltpu.VMEM((PIPE, EMBED_DIM, SUB_W), jnp.float32),
            pltpu.VMEM((EMBED_DIM, B_PER_W), jnp.float32),
            pltpu.VMEM((EMBED_DIM, MAX_TOKENS), jnp.float32),
            pltpu.VMEM((SEQ, B_PER_W), jnp.int32),
            pltpu.VMEM((EMBED_DIM, B_PER_W), jnp.float32),
            pltpu.SemaphoreType.DMA((PIPE,)),
        ],
    )
    return run(title_ids, tokT, tabT, ttabT).T


# speedup vs baseline: 1.0180x; 1.0180x over previous
import jax
import jax.numpy as jnp
from jax import lax
from jax.experimental import pallas as pl
from jax.experimental.pallas import tpu as pltpu
from jax.experimental.pallas import tpu_sc as plsc

VOCAB = 1000000
MAX_TOKENS = 1000
EMBED_DIM = 32
BATCH = 16384
SEQ = 20

NUM_CORES = 2
NUM_SUBCORES = 16
LANES = 16
NUM_WORKERS = NUM_CORES * NUM_SUBCORES
B_PER_W = BATCH // NUM_WORKERS
ROW_CHUNKS = B_PER_W // LANES
TILE_W = 128
SUB_W = 128
PIPE = 8


def _body(tid_hbm, tokT_hbm, tabT_hbm, ttabT_hbm, outT_hbm,
          tid_v, blk_v, titleT_v, ttab_v, tok_v, poolT_v, gsems):
    wid = lax.axis_index("s") * NUM_CORES + lax.axis_index("c")
    base = wid * B_PER_W

    pltpu.sync_copy(tid_hbm.at[pl.ds(base, B_PER_W)], tid_v)
    pltpu.sync_copy(tokT_hbm.at[:, pl.ds(base, B_PER_W)], tok_v)
    pltpu.sync_copy(ttabT_hbm, ttab_v)

    iota = lax.iota(jnp.int32, LANES)
    zeros16 = jnp.zeros((LANES,), jnp.float32)

    plsc.store_scatter(ttab_v, [iota, jnp.zeros((LANES,), jnp.int32)], zeros16)
    plsc.store_scatter(ttab_v, [iota + LANES, jnp.zeros((LANES,), jnp.int32)],
                       zeros16)

    def _col_copy(col8, slot):
        c0 = pl.multiple_of(col8, SUB_W)
        return pltpu.make_async_copy(
            tabT_hbm.at[:, pl.ds(c0, SUB_W)], blk_v.at[slot], gsems.at[slot])

    def title_body(c, carry):
        idvec = tid_v[pl.ds(pl.multiple_of(c * LANES, LANES), LANES)]
        col8 = (idvec // SUB_W) * SUB_W
        off8 = lax.rem(idvec, SUB_W)
        for l in range(PIPE):
            _col_copy(col8[l], l).start()
        for l in range(LANES):
            slot = l % PIPE
            _col_copy(col8[l], slot).wait()
            off = jnp.full((LANES,), off8[l], jnp.int32)
            v0 = plsc.load_gather(blk_v.at[slot], [iota, off])
            v1 = plsc.load_gather(blk_v.at[slot], [iota + LANES, off])
            if l + PIPE < LANES:
                _col_copy(col8[l + PIPE], slot).start()
            icol = jnp.full((LANES,), c * LANES + l, jnp.int32)
            plsc.store_scatter(titleT_v, [iota, icol], v0)
            plsc.store_scatter(titleT_v, [iota + LANES, icol], v1)
        return carry

    lax.fori_loop(0, ROW_CHUNKS, title_body, 0)

    def _tree_sum(vals):
        while len(vals) > 1:
            nxt = [a + b for a, b in zip(vals[::2], vals[1::2])]
            if len(vals) % 2:
                nxt.append(vals[-1])
            vals = nxt
        return vals[0]

    def chunk_body(j, carry):
        row_idx = j * LANES + iota
        tks = [
            plsc.load_gather(tok_v, [jnp.full((LANES,), l, jnp.int32), row_idx])
            for l in range(SEQ)
        ]
        cnt = _tree_sum([(tk != 0).astype(jnp.float32) for tk in tks])
        rcp = 1.0 / jnp.maximum(cnt, 1e-9)
        for d in range(EMBED_DIM):
            drow = jnp.full((LANES,), d, jnp.int32)
            s = _tree_sum([plsc.load_gather(ttab_v, [drow, tk]) for tk in tks])
            plsc.store_scatter(poolT_v, [drow, row_idx], s * rcp)
        return carry

    lax.fori_loop(0, ROW_CHUNKS, chunk_body, 0)

    pltpu.sync_copy(titleT_v, outT_hbm.at[pl.ds(0, EMBED_DIM), pl.ds(base, B_PER_W)])
    pltpu.sync_copy(poolT_v, outT_hbm.at[pl.ds(EMBED_DIM, EMBED_DIM), pl.ds(base, B_PER_W)])


@jax.jit
def kernel(title_ids, token_ids, title_table, token_table):
    tokT = token_ids.T
    tabT = title_table.T
    ttabT = token_table.T

    run = pl.kernel(
        _body,
        out_type=jax.ShapeDtypeStruct((2 * EMBED_DIM, BATCH), jnp.float32),
        mesh=plsc.VectorSubcoreMesh(core_axis_name="c", subcore_axis_name="s"),
        compiler_params=pltpu.CompilerParams(
            needs_layout_passes=False, use_tc_tiling_on_sc=True),
        scratch_types=[
            pltpu.VMEM((B_PER_W,), jnp.int32),
            pltpu.VMEM((PIPE, EMBED_DIM, SUB_W), jnp.float32),
            pltpu.VMEM((EMBED_DIM, B_PER_W), jnp.float32),
            pltpu.VMEM((EMBED_DIM, MAX_TOKENS), jnp.float32),
            pltpu.VMEM((SEQ, B_PER_W), jnp.int32),
            pltpu.VMEM((EMBED_DIM, B_PER_W), jnp.float32),
            pltpu.SemaphoreType.DMA((PIPE,)),
        ],
    )
    return run(title_ids, tokT, tabT, ttabT).T
